# baseline (speedup 1.0000x reference)
import jax
import jax.numpy as jnp
from jax.experimental import pallas as pl
from jax.experimental.pallas import tpu as pltpu

N = 10000
BM = 400


def _nan2num(v):
    return jnp.nan_to_num(v, nan=0.0, posinf=1.0, neginf=0.0)


def _layer1_body(x_ref, w1t_ref, b1_ref, w2t_ref, b2_ref, adj_ref, b_ref,
                 a_scr):
    @pl.when(pl.program_id(0) == 0)
    def _():
        xs = _nan2num(x_ref[...])
        a_scr[...] = (
            jnp.dot(xs, w1t_ref[...], preferred_element_type=jnp.float32)
            + b1_ref[...])

    acc = jnp.dot(adj_ref[...], a_scr[...],
                  preferred_element_type=jnp.float32)
    h1 = jnp.maximum(_nan2num(acc), 0.0)
    b_ref[...] = (
        jnp.dot(h1, w2t_ref[...], preferred_element_type=jnp.float32)
        + b2_ref[...])


def _layer2_body(bfull_ref, adj_ref, out_ref):
    acc = jnp.dot(adj_ref[...], bfull_ref[...],
                  preferred_element_type=jnp.float32)
    out_ref[...] = _nan2num(acc)


@jax.jit
def kernel(x, init_adj, W1, b1, W2, b2):
    d_in = x.shape[1]
    d_hid = W1.shape[0]
    d_out = W2.shape[0]
    w1t = W1.T
    w2t = W2.T
    b1r = b1.reshape(1, d_hid)
    b2r = b2.reshape(1, d_out)

    grid = (N // BM,)

    b_mat = pl.pallas_call(
        _layer1_body,
        grid=grid,
        in_specs=[
            pl.BlockSpec((N, d_in), lambda i: (0, 0)),
            pl.BlockSpec((d_in, d_hid), lambda i: (0, 0)),
            pl.BlockSpec((1, d_hid), lambda i: (0, 0)),
            pl.BlockSpec((d_hid, d_out), lambda i: (0, 0)),
            pl.BlockSpec((1, d_out), lambda i: (0, 0)),
            pl.BlockSpec((BM, N), lambda i: (i, 0)),
        ],
        out_specs=pl.BlockSpec((BM, d_out), lambda i: (i, 0)),
        out_shape=jax.ShapeDtypeStruct((N, d_out), jnp.float32),
        scratch_shapes=[pltpu.VMEM((N, d_hid), jnp.float32)],
    )(x, w1t, b1r, w2t, b2r, init_adj)

    out = pl.pallas_call(
        _layer2_body,
        grid=grid,
        in_specs=[
            pl.BlockSpec((N, d_out), lambda i: (0, 0)),
            pl.BlockSpec((BM, N), lambda i: (i, 0)),
        ],
        out_specs=pl.BlockSpec((BM, d_out), lambda i: (i, 0)),
        out_shape=jax.ShapeDtypeStruct((N, d_out), jnp.float32),
    )(b_mat, init_adj)

    return out


# speedup vs baseline: 1.0332x; 1.0332x over previous
import jax
import jax.numpy as jnp
from jax.experimental import pallas as pl
from jax.experimental.pallas import tpu as pltpu

N = 10000
BM = 400


def _nan2num(v):
    return jnp.nan_to_num(v, nan=0.0, posinf=1.0, neginf=0.0)


def _gcn_body(x_ref, w1t_ref, b1_ref, w2t_ref, b2_ref, adj_ref, out_ref,
              a_scr, b_scr):
    p = pl.program_id(0)
    i = pl.program_id(1)

    @pl.when((p == 0) & (i == 0))
    def _():
        xs = _nan2num(x_ref[...])
        a_scr[...] = (
            jnp.dot(xs, w1t_ref[...], preferred_element_type=jnp.float32)
            + b1_ref[...])

    @pl.when(p == 0)
    def _():
        acc = jnp.dot(adj_ref[...], a_scr[...],
                      preferred_element_type=jnp.float32)
        h1 = jnp.maximum(_nan2num(acc), 0.0)
        b_scr[pl.ds(i * BM, BM), :] = (
            jnp.dot(h1, w2t_ref[...], preferred_element_type=jnp.float32)
            + b2_ref[...])

    @pl.when(p == 1)
    def _():
        acc = jnp.dot(adj_ref[...], b_scr[...],
                      preferred_element_type=jnp.float32)
        out_ref[...] = _nan2num(acc)


@jax.jit
def kernel(x, init_adj, W1, b1, W2, b2):
    d_in = x.shape[1]
    d_hid = W1.shape[0]
    d_out = W2.shape[0]
    w1t = W1.T
    w2t = W2.T
    b1r = b1.reshape(1, d_hid)
    b2r = b2.reshape(1, d_out)

    num_i = N // BM

    out = pl.pallas_call(
        _gcn_body,
        grid=(2, num_i),
        in_specs=[
            pl.BlockSpec((N, d_in), lambda p, i: (0, 0)),
            pl.BlockSpec((d_in, d_hid), lambda p, i: (0, 0)),
            pl.BlockSpec((1, d_hid), lambda p, i: (0, 0)),
            pl.BlockSpec((d_hid, d_out), lambda p, i: (0, 0)),
            pl.BlockSpec((1, d_out), lambda p, i: (0, 0)),
            pl.BlockSpec((BM, N), lambda p, i: (i, 0)),
        ],
        out_specs=pl.BlockSpec(
            (BM, d_out), lambda p, i: (jnp.where(p == 1, i, 0), 0)),
        out_shape=jax.ShapeDtypeStruct((N, d_out), jnp.float32),
        scratch_shapes=[
            pltpu.VMEM((N, d_hid), jnp.float32),
            pltpu.VMEM((N, d_out), jnp.float32),
        ],
    )(x, w1t, b1r, w2t, b2r, init_adj)

    return out


# speedup vs baseline: 1.0381x; 1.0047x over previous
import jax
import jax.numpy as jnp
from jax.experimental import pallas as pl
from jax.experimental.pallas import tpu as pltpu

N = 10000
BM = 200
NUM_I = N // BM
P = 9
NUM1 = NUM_I - P


def _nan2num(v):
    return jnp.nan_to_num(v, nan=0.0, posinf=1.0, neginf=0.0)


def _prep_body(x_ref, w1t_ref, b1_ref, a_ref):
    xs = _nan2num(x_ref[...])
    a_ref[...] = (
        jnp.dot(xs, w1t_ref[...], preferred_element_type=jnp.float32)
        + b1_ref[...])


def _gcn_body(a_ref, w2t_ref, b2_ref, adj_ref, out1_ref, out2_ref,
              b_scr, bbf_scr, cache_scr):
    s = pl.program_id(0)

    @pl.when(s < NUM_I)
    def _():
        acc = jnp.dot(adj_ref[...], a_ref[...],
                      preferred_element_type=jnp.float32)
        h1 = jnp.maximum(_nan2num(acc), 0.0)
        b_scr[pl.ds(s * BM, BM), :] = (
            jnp.dot(h1, w2t_ref[...], preferred_element_type=jnp.float32)
            + b2_ref[...])

    @pl.when((s >= NUM_I - P) & (s < NUM_I))
    def _():
        cache_scr[s - (NUM_I - P)] = adj_ref[...].astype(jnp.bfloat16)

    @pl.when(s == NUM_I)
    def _():
        bbf_scr[...] = b_scr[...].astype(jnp.bfloat16)

    @pl.when(s >= NUM_I)
    def _():
        acc = jnp.dot(adj_ref[...], b_scr[...],
                      preferred_element_type=jnp.float32)
        out1_ref[...] = _nan2num(acc)

    @pl.when((s >= NUM_I) & (s < NUM_I + P))
    def _():
        acc = jnp.dot(cache_scr[s - NUM_I], bbf_scr[...],
                      preferred_element_type=jnp.float32)
        out2_ref[...] = _nan2num(acc)


@jax.jit
def kernel(x, init_adj, W1, b1, W2, b2):
    d_in = x.shape[1]
    d_hid = W1.shape[0]
    d_out = W2.shape[0]
    w1t = W1.T
    w2t = W2.T
    b1r = b1.reshape(1, d_hid)
    b2r = b2.reshape(1, d_out)

    a_mat = pl.pallas_call(
        _prep_body,
        out_shape=jax.ShapeDtypeStruct((N, d_hid), jnp.float32),
    )(x, w1t, b1r)

    out1, out2 = pl.pallas_call(
        _gcn_body,
        grid=(NUM_I + NUM1,),
        in_specs=[
            pl.BlockSpec((N, d_hid), lambda s: (0, 0)),
            pl.BlockSpec((d_hid, d_out), lambda s: (0, 0)),
            pl.BlockSpec((1, d_out), lambda s: (0, 0)),
            pl.BlockSpec((BM, N),
                         lambda s: (jnp.where(s < NUM_I, s, s - NUM_I), 0)),
        ],
        out_specs=[
            pl.BlockSpec((BM, d_out),
                         lambda s: (jnp.where(s < NUM_I, 0, s - NUM_I), 0)),
            pl.BlockSpec((BM, d_out),
                         lambda s: (jnp.clip(s - NUM_I, 0, P - 1), 0)),
        ],
        out_shape=[
            jax.ShapeDtypeStruct((NUM1 * BM, d_out), jnp.float32),
            jax.ShapeDtypeStruct((P * BM, d_out), jnp.float32),
        ],
        scratch_shapes=[
            pltpu.VMEM((N, d_out), jnp.float32),
            pltpu.VMEM((N, d_out), jnp.bfloat16),
            pltpu.VMEM((P, BM, N), jnp.bfloat16),
        ],
        compiler_params=pltpu.CompilerParams(
            vmem_limit_bytes=64 * 1024 * 1024),
    )(a_mat, w2t, b2r, init_adj)

    return jnp.concatenate([out1, out2], axis=0)
